# baseline (speedup 1.0000x reference)
import functools

import jax
import jax.numpy as jnp
from jax import lax
from jax.experimental import pallas as pl
from jax.experimental.pallas import tpu as pltpu
from jax.experimental.pallas import tpu_sc as plsc

NUM_EMB = 8192
DIM = 1024
B = 4 * 8192

_info = plsc.get_sparse_core_info()
_NC = _info.num_cores
_NS = _info.num_subcores
NW = _NC * _NS
BPW = B // NW
C = 40
_SIZES = [C] * (BPW // C) + ([BPW % C] if BPW % C else [])
_OFFS = [sum(_SIZES[:i]) for i in range(len(_SIZES))]
NCHUNK = len(_SIZES)
NB = 3


def _emb_body(x_hbm, w_hbm, out_hbm, ic0, ic1, ic2, buf0, buf1, buf2,
              si0, si1, si2, sg0, sg1, sg2, so0, so1, so2):
    wid = lax.axis_index("s") * _NC + lax.axis_index("c")
    base = wid * BPW

    ics = (ic0, ic1, ic2)
    bufs = (buf0, buf1, buf2)
    sis = (si0, si1, si2)
    sgs = (sg0, sg1, sg2)
    sos = (so0, so1, so2)
    last_put = [None, None, None]

    def idx_start(g, b):
        n = _SIZES[g]
        pltpu.make_async_copy(
            x_hbm.at[pl.ds(base + _OFFS[g], n)],
            ics[b].at[pl.ds(0, n)], sis[b]
        ).start()

    def idx_wait(g, b):
        n = _SIZES[g]
        pltpu.make_async_copy(
            x_hbm.at[pl.ds(base, n)], ics[b].at[pl.ds(0, n)], sis[b]
        ).wait()

    def gather_start(b):
        pltpu.make_async_copy(w_hbm.at[ics[b]], bufs[b], sgs[b]).start()

    def gather_wait(b):
        pltpu.make_async_copy(w_hbm.at[ics[b]], bufs[b], sgs[b]).wait()

    def put_start(g, b):
        n = _SIZES[g]
        pltpu.make_async_copy(
            bufs[b].at[pl.ds(0, n)],
            out_hbm.at[pl.ds(base + _OFFS[g], n)], sos[b]
        ).start()
        last_put[b] = g

    def put_wait(b):
        n = _SIZES[last_put[b]]
        pltpu.make_async_copy(
            bufs[b].at[pl.ds(0, n)],
            out_hbm.at[pl.ds(base, n)], sos[b]
        ).wait()

    idx_start(0, 0)
    idx_start(1, 1)
    idx_start(2, 2)
    idx_wait(0, 0)
    gather_start(0)
    idx_wait(1, 1)
    gather_start(1)
    for g in range(NCHUNK):
        b = g % NB
        gather_wait(b)
        if g + NB < NCHUNK:
            idx_start(g + NB, b)
        put_start(g, b)
        h = g + 2
        if h < NCHUNK:
            if h >= NB:
                put_wait(h % NB)
            idx_wait(h, h % NB)
            gather_start(h % NB)
    for k in range(NCHUNK - 3, NCHUNK):
        put_wait(k % NB)


@jax.jit
def _emb(x_flat, weight):
    mesh = plsc.VectorSubcoreMesh(core_axis_name="c", subcore_axis_name="s")
    fn = functools.partial(
        pl.kernel,
        mesh=mesh,
        out_type=jax.ShapeDtypeStruct((B, DIM), jnp.float32),
        scratch_types=[
            pltpu.VMEM((C,), jnp.int32),
            pltpu.VMEM((C,), jnp.int32),
            pltpu.VMEM((C,), jnp.int32),
            pltpu.VMEM((C, DIM), jnp.float32),
            pltpu.VMEM((C, DIM), jnp.float32),
            pltpu.VMEM((C, DIM), jnp.float32),
            pltpu.SemaphoreType.DMA,
            pltpu.SemaphoreType.DMA,
            pltpu.SemaphoreType.DMA,
            pltpu.SemaphoreType.DMA,
            pltpu.SemaphoreType.DMA,
            pltpu.SemaphoreType.DMA,
            pltpu.SemaphoreType.DMA,
            pltpu.SemaphoreType.DMA,
            pltpu.SemaphoreType.DMA,
        ],
    )(_emb_body)
    return fn(x_flat, weight)


def kernel(x, weight):
    out = _emb(x.reshape(-1), weight)
    return out.reshape(x.shape + (weight.shape[1],))


# speedup vs baseline: 1.0122x; 1.0122x over previous
import functools

import jax
import jax.numpy as jnp
from jax import lax
from jax.experimental import pallas as pl
from jax.experimental.pallas import tpu as pltpu
from jax.experimental.pallas import tpu_sc as plsc

NUM_EMB = 8192
DIM = 1024
B = 4 * 8192

_info = plsc.get_sparse_core_info()
_NC = _info.num_cores
_NS = _info.num_subcores
NW = _NC * _NS
BPW = B // NW
C = 24
NB = 5
_SIZES = [C] * (BPW // C) + ([BPW % C] if BPW % C else [])
_OFFS = [sum(_SIZES[:i]) for i in range(len(_SIZES))]
NCHUNK = len(_SIZES)
_LEAD = NB - 1


def _emb_body(x_hbm, w_hbm, out_hbm, idx_v, bufs, sgs, sos):
    wid = lax.axis_index("s") * _NC + lax.axis_index("c")
    base = wid * BPW
    _head = min(_LEAD * C, BPW)
    pltpu.sync_copy(x_hbm.at[pl.ds(base, _head)], idx_v.at[pl.ds(0, _head)])

    last_put = [None] * NB

    def gather_start(g, b):
        n = _SIZES[g]
        pltpu.make_async_copy(
            w_hbm.at[idx_v.at[pl.ds(_OFFS[g], n)]],
            bufs[b].at[pl.ds(0, n)], sgs[b]
        ).start()

    def gather_wait(g, b):
        n = _SIZES[g]
        pltpu.make_async_copy(
            w_hbm.at[idx_v.at[pl.ds(0, n)]],
            bufs[b].at[pl.ds(0, n)], sgs[b]
        ).wait()

    def put_start(g, b):
        n = _SIZES[g]
        pltpu.make_async_copy(
            bufs[b].at[pl.ds(0, n)],
            out_hbm.at[pl.ds(base + _OFFS[g], n)], sos[b]
        ).start()
        last_put[b] = g

    def put_wait(b):
        n = _SIZES[last_put[b]]
        pltpu.make_async_copy(
            bufs[b].at[pl.ds(0, n)],
            out_hbm.at[pl.ds(base, n)], sos[b]
        ).wait()

    for g in range(_LEAD):
        gather_start(g, g % NB)
    pltpu.sync_copy(
        x_hbm.at[pl.ds(base + _head, BPW - _head)],
        idx_v.at[pl.ds(_head, BPW - _head)],
    )
    for g in range(NCHUNK):
        b = g % NB
        gather_wait(g, b)
        put_start(g, b)
        h = g + _LEAD
        if h < NCHUNK:
            if h >= NB:
                put_wait(h % NB)
            gather_start(h, h % NB)
    for k in range(NCHUNK - NB, NCHUNK):
        put_wait(k % NB)


def _emb_entry(x_hbm, w_hbm, out_hbm, idx_v, b0, b1, b2, b3, b4,
               g0, g1, g2, g3, g4, o0, o1, o2, o3, o4):
    _emb_body(x_hbm, w_hbm, out_hbm, idx_v,
              (b0, b1, b2, b3, b4),
              (g0, g1, g2, g3, g4),
              (o0, o1, o2, o3, o4))


@jax.jit
def _emb(x_flat, weight):
    mesh = plsc.VectorSubcoreMesh(core_axis_name="c", subcore_axis_name="s")
    fn = functools.partial(
        pl.kernel,
        mesh=mesh,
        out_type=jax.ShapeDtypeStruct((B, DIM), jnp.float32),
        scratch_types=(
            [pltpu.VMEM((BPW,), jnp.int32)]
            + [pltpu.VMEM((C, DIM), jnp.float32)] * NB
            + [pltpu.SemaphoreType.DMA] * (2 * NB)
        ),
    )(_emb_entry)
    return fn(x_flat, weight)


def kernel(x, weight):
    out = _emb(x.reshape(-1), weight)
    return out.reshape(x.shape + (weight.shape[1],))
